# Initial kernel scaffold; baseline (speedup 1.0000x reference)
import jax
import jax.numpy as jnp
from jax.experimental import pallas as pl


def kernel(x, edge_index, W, b):
    raise NotImplementedError("write your pallas kernel here")


# speedup vs baseline: 11.3437x; 11.3437x over previous
import functools

import jax
import jax.numpy as jnp
from jax import lax
from jax.experimental import pallas as pl
from jax.experimental.pallas import tpu as pltpu
from jax.experimental.pallas import tpu_sc as plsc

NC, NS, L = 2, 16, 16
NW = NC * NS


def _deg_kernel(N, E, RB):
    EW = E // NW
    nfull, rem = divmod(EW, L)
    NB = N // RB
    mesh = plsc.VectorSubcoreMesh(core_axis_name="c", subcore_axis_name="s")

    @functools.partial(
        pl.kernel,
        out_type=jax.ShapeDtypeStruct((NB, NW, RB), jnp.float32),
        mesh=mesh,
        scratch_types=[
            pltpu.VMEM((EW + L,), jnp.int32),
            pltpu.VMEM((N,), jnp.float32),
        ],
        compiler_params=pltpu.CompilerParams(needs_layout_passes=False, use_tc_tiling_on_sc=False),
    )
    def deg(dst_hbm, degp_hbm, dst_v, hist):
        c = lax.axis_index("c")
        s = lax.axis_index("s")
        w = s * NC + c

        def zero_one(i, carry):
            hist[pl.ds(i * L, L)] = jnp.zeros((L,), jnp.float32)
            return carry

        lax.fori_loop(0, N // L, zero_one, 0)

        base = w * EW
        if rem:
            dst_v[pl.ds(nfull * L, L)] = jnp.zeros((L,), jnp.int32)
        pltpu.sync_copy(dst_hbm.at[pl.ds(base, EW)], dst_v.at[pl.ds(0, EW)])

        ones = jnp.ones((L,), jnp.float32)
        lanes = lax.iota(jnp.int32, L)

        def add_one(i, carry):
            idx = dst_v[pl.ds(i * L, L)]
            plsc.addupdate_scatter(hist, [idx], ones)
            return carry

        lax.fori_loop(0, nfull, add_one, 0)
        if rem:
            idx = dst_v[pl.ds(nfull * L, L)]
            plsc.addupdate_scatter(hist, [idx], ones, mask=lanes < rem)
        for k in range(NB):
            pltpu.sync_copy(hist.at[pl.ds(k * RB, RB)], degp_hbm.at[k, w])

    return deg


def _matmul_kernel(N, C, RB):
    H = C // 2

    def body(x_ref, w_ref, degp_ref, y_ref):
        deg = jnp.sum(degp_ref[0], axis=0) + 1.0
        dis = lax.rsqrt(deg)
        xw = jnp.dot(x_ref[...], w_ref[...], preferred_element_type=jnp.float32)
        y = xw * dis[:, None]
        y_ref[0] = y[:, :H]
        y_ref[1] = y[:, H:]

    return pl.pallas_call(
        body,
        grid=(N // RB,),
        in_specs=[
            pl.BlockSpec((RB, C), lambda i: (i, 0)),
            pl.BlockSpec((C, C), lambda i: (0, 0)),
            pl.BlockSpec((1, NW, RB), lambda i: (i, 0, 0)),
        ],
        out_specs=pl.BlockSpec((2, RB, H), lambda i: (0, i, 0)),
        out_shape=jax.ShapeDtypeStruct((2, N, H), jnp.float32),
    )


def _scatter_kernel(N, H, E, K, RI):
    EW = E // NS
    RT = N // NS
    mesh = plsc.VectorSubcoreMesh(core_axis_name="c", subcore_axis_name="s")

    @functools.partial(
        pl.kernel,
        out_type=jax.ShapeDtypeStruct((2, N, H), jnp.float32),
        mesh=mesh,
        scratch_types=[
            pltpu.VMEM((RI, H), jnp.float32),
            pltpu.VMEM((K,), jnp.int32),
            pltpu.VMEM((K,), jnp.int32),
            pltpu.VMEM((K, H), jnp.float32),
            pltpu.VMEM_SHARED((N, H), jnp.float32),
            pltpu.SemaphoreType.DMA,
        ],
        compiler_params=pltpu.CompilerParams(needs_layout_passes=False, use_tc_tiling_on_sc=False),
    )
    def scatter(y_hbm, src_hbm, dst_hbm, acc_hbm, ibuf, sidx, didx, gbuf,
                acc_sh, sem):
        c = lax.axis_index("c")
        s = lax.axis_index("s")
        row0 = s * RT

        def init_one(k, carry):
            r = row0 + k * RI
            pltpu.sync_copy(y_hbm.at[c].at[pl.ds(r, RI)], ibuf)
            pltpu.sync_copy(ibuf, acc_sh.at[pl.ds(r, RI)])
            return carry

        lax.fori_loop(0, RT // RI, init_one, 0)
        plsc.subcore_barrier()

        def edge_chunk(i, carry):
            base = s * EW + i * K
            pltpu.sync_copy(src_hbm.at[pl.ds(base, K)], sidx)
            pltpu.sync_copy(dst_hbm.at[pl.ds(base, K)], didx)
            pltpu.async_copy(y_hbm.at[c].at[sidx], gbuf, sem).wait()
            pltpu.sync_copy(gbuf, acc_sh.at[didx], add=True)
            return carry

        lax.fori_loop(0, EW // K, edge_chunk, 0)
        plsc.subcore_barrier()

        def out_one(k, carry):
            r = row0 + k * RI
            pltpu.sync_copy(acc_sh.at[pl.ds(r, RI)], ibuf)
            pltpu.sync_copy(ibuf, acc_hbm.at[c].at[pl.ds(r, RI)])
            return carry

        lax.fori_loop(0, RT // RI, out_one, 0)

    return scatter


def _finish_kernel(N, C, RB):
    def body(acc_ref, degp_ref, b_ref, o_ref):
        deg = jnp.sum(degp_ref[0], axis=0) + 1.0
        dis = lax.rsqrt(deg)
        cat = jnp.concatenate([acc_ref[0], acc_ref[1]], axis=1)
        o_ref[...] = jnp.maximum(cat * dis[:, None] + b_ref[...], 0.0)

    return pl.pallas_call(
        body,
        grid=(N // RB,),
        in_specs=[
            pl.BlockSpec((2, RB, C // 2), lambda i: (0, i, 0)),
            pl.BlockSpec((1, NW, RB), lambda i: (i, 0, 0)),
            pl.BlockSpec((1, C), lambda i: (0, 0)),
        ],
        out_specs=pl.BlockSpec((RB, C), lambda i: (i, 0)),
        out_shape=jax.ShapeDtypeStruct((N, C), jnp.float32),
    )


def kernel(x, edge_index, W, b):
    N, C = x.shape
    E = edge_index.shape[1]
    H = C // 2
    RB = 1000

    src = edge_index[0].astype(jnp.int32)
    dst = edge_index[1].astype(jnp.int32)

    degp = _deg_kernel(N, E, RB)(dst)
    y = _matmul_kernel(N, C, RB)(x, W, degp)
    acc = _scatter_kernel(N, H, E, 80, 125)(y, src, dst)
    return _finish_kernel(N, C, RB)(acc, degp, b.reshape(1, C))
